# baseline (device time: 357299 ns/iter reference)
import jax
import jax.numpy as jnp
from jax import lax
from jax.experimental import pallas as pl
from jax.experimental.pallas import tpu as pltpu

N_DEV = 4
N_SUB = 1024
N_CHAIN = 2
COMM_DTYPE = jnp.bfloat16


def kernel(x, w_mat, scale_x, scale_w):
    m_total, _k = x.shape
    _k2, n = w_mat.shape
    m_per = m_total // N_DEV
    half = n // 2
    nb = half // N_SUB
    hops_per_ring = (nb // N_CHAIN) * (N_DEV - 1)

    x = x.astype(jnp.float8_e4m3fn)
    w_mat = w_mat.astype(jnp.float8_e5m2)
    scale = (scale_x * scale_w).reshape(1, 1)

    def body(x_ref, w_ref, scale_ref, out_ref,
             comm0, comm1, comm2, comm3, stage,
             send_sems, recv_sems, out_sem,
             ready0, ready1, ready2, ready3):
        d = lax.axis_index("i")
        right = lax.rem(d + 1, N_DEV)
        left = lax.rem(d + N_DEV - 1, N_DEV)

        barrier_sem = pltpu.get_barrier_semaphore()
        for nbr in (left, right):
            pl.semaphore_signal(barrier_sem, inc=1, device_id=(nbr,),
                                device_id_type=pl.DeviceIdType.MESH)
        pl.semaphore_wait(barrier_sem, 2)

        def partial(chunk_idx, col_off):
            xc = x_ref[pl.ds(chunk_idx * m_per, m_per), :]
            wc = w_ref[:, pl.ds(col_off, N_SUB)]
            return lax.dot_general(xc, wc, (((1,), (0,)), ((), ())),
                                   preferred_element_type=jnp.float32)

        dir_specs = [
            dict(tgt=right, upstream=left, base=0,
                 seed_c=lax.rem(d + N_DEV - 1, N_DEV),
                 in_c=lambda s: lax.rem(d + 2 * N_DEV - 2 - s, N_DEV)),
            dict(tgt=left, upstream=right, base=half,
                 seed_c=lax.rem(d + 1, N_DEV),
                 in_c=lambda s: lax.rem(d + 2 + s, N_DEV)),
        ]
        comms = [comm0, comm1, comm2, comm3]
        readys = [ready0, ready1, ready2, ready3]
        out_copies = []

        rings = []
        for di, ds_ in enumerate(dir_specs):
            for ch in range(N_CHAIN):
                r = di * N_CHAIN + ch
                blocks = list(range(ch, nb, N_CHAIN))
                rings.append(dict(
                    ds_, idx=r, dir=di, comm=comms[r], ready=readys[r],
                    tokens=[(b, s) for b in blocks for s in range(N_DEV - 1)],
                    H=0, rdma=None, p=None,
                ))
                pl.semaphore_signal(readys[r], inc=1,
                                    device_id=(ds_["upstream"],),
                                    device_id_type=pl.DeviceIdType.MESH)

        rings = [rings[0], rings[2], rings[1], rings[3]]

        def col(ring, blk):
            return ring["base"] + blk * N_SUB

        def seed(ring, blk):
            ring["comm"][ring["H"] % 2] = partial(
                ring["seed_c"], col(ring, blk)).astype(COMM_DTYPE)

        def start(ring):
            H = ring["H"]
            pl.semaphore_wait(ring["ready"], 1)
            rdma = pltpu.make_async_remote_copy(
                src_ref=ring["comm"].at[H % 2],
                dst_ref=ring["comm"].at[(H + 1) % 2],
                send_sem=send_sems.at[ring["idx"], H % 2],
                recv_sem=recv_sems.at[ring["idx"], (H + 1) % 2],
                device_id=(ring["tgt"],),
                device_id_type=pl.DeviceIdType.MESH,
            )
            rdma.start()
            ring["rdma"] = rdma

        def stash(ring, blk, s):
            ring["p"] = partial(ring["in_c"](s), col(ring, blk)).astype(
                COMM_DTYPE)

        def consume(ring, blk, s):
            H = ring["H"]
            ring["rdma"].wait_recv()
            if s < N_DEV - 2:
                ring["comm"][(H + 1) % 2] = (
                    ring["comm"][(H + 1) % 2] + ring["p"])
            else:
                acc = (ring["comm"][(H + 1) % 2].astype(jnp.float32)
                       + ring["p"].astype(jnp.float32))
                if out_copies:
                    out_copies[-1].wait()
                stage[...] = jnp.maximum(acc * scale_ref[0, 0], 0.0)
                cp = pltpu.make_async_copy(
                    stage,
                    out_ref.at[:, pl.ds(col(ring, blk), N_SUB)],
                    out_sem,
                )
                cp.start()
                out_copies.append(cp)
            ring["H"] = H + 1

        for ring in rings:
            blk, s = ring["tokens"][0]
            seed(ring, blk)
            start(ring)
            stash(ring, blk, s)
        for t in range(hops_per_ring):
            for ring in rings:
                ring["rdma"].wait_send()
                if t + 1 < hops_per_ring:
                    pl.semaphore_signal(ring["ready"], inc=1,
                                        device_id=(ring["upstream"],),
                                        device_id_type=pl.DeviceIdType.MESH)
            for ring in rings:
                blk, s = ring["tokens"][t]
                consume(ring, blk, s)
            if t + 1 < hops_per_ring:
                for ring in rings:
                    nblk, ns = ring["tokens"][t + 1]
                    if ns == 0:
                        seed(ring, nblk)
                    start(ring)
                for ring in rings:
                    nblk, ns = ring["tokens"][t + 1]
                    stash(ring, nblk, ns)
        out_copies[-1].wait()

    return pl.pallas_call(
        body,
        out_shape=jax.ShapeDtypeStruct((m_per, n), jnp.float32),
        in_specs=[
            pl.BlockSpec(memory_space=pltpu.VMEM),
            pl.BlockSpec(memory_space=pltpu.VMEM),
            pl.BlockSpec(memory_space=pltpu.SMEM),
        ],
        out_specs=pl.BlockSpec(memory_space=pl.ANY),
        scratch_shapes=[
            pltpu.VMEM((2, m_per, N_SUB), COMM_DTYPE),
            pltpu.VMEM((2, m_per, N_SUB), COMM_DTYPE),
            pltpu.VMEM((2, m_per, N_SUB), COMM_DTYPE),
            pltpu.VMEM((2, m_per, N_SUB), COMM_DTYPE),
            pltpu.VMEM((m_per, N_SUB), jnp.float32),
            pltpu.SemaphoreType.DMA((4, 2)),
            pltpu.SemaphoreType.DMA((4, 2)),
            pltpu.SemaphoreType.DMA,
            pltpu.SemaphoreType.REGULAR,
            pltpu.SemaphoreType.REGULAR,
            pltpu.SemaphoreType.REGULAR,
            pltpu.SemaphoreType.REGULAR,
        ],
        compiler_params=pltpu.CompilerParams(
            collective_id=0, vmem_limit_bytes=50 * 1024 * 1024),
    )(x, w_mat, scale)


# device time: 353741 ns/iter; 1.0101x vs baseline; 1.0101x over previous
import jax
import jax.numpy as jnp
from jax import lax
from jax.experimental import pallas as pl
from jax.experimental.pallas import tpu as pltpu

N_DEV = 4
N_SUB = 1024
N_CHAIN = 2
COMM_DTYPE = jnp.bfloat16


def kernel(x, w_mat, scale_x, scale_w):
    m_total, _k = x.shape
    _k2, n = w_mat.shape
    m_per = m_total // N_DEV
    half = n // 2
    nb = half // N_SUB
    hops_per_ring = (nb // N_CHAIN) * (N_DEV - 1)

    x = x.astype(jnp.float8_e4m3fn)
    w_mat = w_mat.astype(jnp.float8_e5m2)
    scale = (scale_x * scale_w).reshape(1, 1)

    def body(x_ref, w_ref, scale_ref, out_ref,
             comm0, comm1, comm2, comm3, stage,
             send_sems, recv_sems, out_sem,
             ready0, ready1, ready2, ready3):
        d = lax.axis_index("i")
        right = lax.rem(d + 1, N_DEV)
        left = lax.rem(d + N_DEV - 1, N_DEV)

        barrier_sem = pltpu.get_barrier_semaphore()
        for nbr in (left, right):
            pl.semaphore_signal(barrier_sem, inc=1, device_id=(nbr,),
                                device_id_type=pl.DeviceIdType.MESH)
        pl.semaphore_wait(barrier_sem, 2)

        def partial(chunk_idx, col_off):
            xc = x_ref[pl.ds(chunk_idx * m_per, m_per), :]
            wc = w_ref[:, pl.ds(col_off, N_SUB)]
            return lax.dot_general(xc, wc, (((1,), (0,)), ((), ())),
                                   preferred_element_type=jnp.float32)

        dir_specs = [
            dict(tgt=right, upstream=left, base=0,
                 seed_c=lax.rem(d + N_DEV - 1, N_DEV),
                 in_c=lambda s: lax.rem(d + 2 * N_DEV - 2 - s, N_DEV)),
            dict(tgt=left, upstream=right, base=half,
                 seed_c=lax.rem(d + 1, N_DEV),
                 in_c=lambda s: lax.rem(d + 2 + s, N_DEV)),
        ]
        comms = [comm0, comm1, comm2, comm3]
        readys = [ready0, ready1, ready2, ready3]
        out_copies = []

        rings = []
        for di, ds_ in enumerate(dir_specs):
            for ch in range(N_CHAIN):
                r = di * N_CHAIN + ch
                blocks = list(range(ch, nb, N_CHAIN))
                rings.append(dict(
                    ds_, idx=r, dir=di, comm=comms[r], ready=readys[r],
                    tokens=[(b, s) for b in blocks for s in range(N_DEV - 1)],
                    H=0, rdma=None, p=None,
                ))
                pl.semaphore_signal(readys[r], inc=1,
                                    device_id=(ds_["upstream"],),
                                    device_id_type=pl.DeviceIdType.MESH)

        rings = [rings[0], rings[2], rings[1], rings[3]]

        def col(ring, blk):
            return ring["base"] + blk * N_SUB

        def seed(ring, blk):
            ring["comm"][ring["H"] % 2] = partial(
                ring["seed_c"], col(ring, blk)).astype(COMM_DTYPE)

        def start(ring):
            H = ring["H"]
            pl.semaphore_wait(ring["ready"], 1)
            rdma = pltpu.make_async_remote_copy(
                src_ref=ring["comm"].at[H % 2],
                dst_ref=ring["comm"].at[(H + 1) % 2],
                send_sem=send_sems.at[ring["idx"], H % 2],
                recv_sem=recv_sems.at[ring["idx"], (H + 1) % 2],
                device_id=(ring["tgt"],),
                device_id_type=pl.DeviceIdType.MESH,
            )
            rdma.start()
            ring["rdma"] = rdma

        def stash(ring, blk, s):
            ring["p"] = partial(ring["in_c"](s), col(ring, blk)).astype(
                COMM_DTYPE)

        def consume(ring, blk, s):
            H = ring["H"]
            ring["rdma"].wait_recv()
            if s < N_DEV - 2:
                ring["comm"][(H + 1) % 2] = (
                    ring["comm"][(H + 1) % 2] + ring["p"])
            else:
                acc = (ring["comm"][(H + 1) % 2].astype(jnp.float32)
                       + ring["p"].astype(jnp.float32))
                if out_copies:
                    out_copies[-1].wait()
                stage[...] = jnp.maximum(acc * scale_ref[0, 0], 0.0)
                cp = pltpu.make_async_copy(
                    stage,
                    out_ref.at[:, pl.ds(col(ring, blk), N_SUB)],
                    out_sem,
                )
                cp.start()
                out_copies.append(cp)
            ring["H"] = H + 1

        for ring in rings:
            blk, s = ring["tokens"][0]
            seed(ring, blk)
            start(ring)
            stash(ring, blk, s)
        for t in range(hops_per_ring):
            for ring in rings:
                ring["rdma"].wait_send()
                if t + 1 < hops_per_ring:
                    pl.semaphore_signal(ring["ready"], inc=1,
                                        device_id=(ring["upstream"],),
                                        device_id_type=pl.DeviceIdType.MESH)
            for ring in rings:
                blk, s = ring["tokens"][t]
                consume(ring, blk, s)
                if t + 1 < hops_per_ring:
                    nblk, ns = ring["tokens"][t + 1]
                    if ns == 0:
                        seed(ring, nblk)
                    start(ring)
                    stash(ring, nblk, ns)
        out_copies[-1].wait()

    return pl.pallas_call(
        body,
        out_shape=jax.ShapeDtypeStruct((m_per, n), jnp.float32),
        in_specs=[
            pl.BlockSpec(memory_space=pltpu.VMEM),
            pl.BlockSpec(memory_space=pltpu.VMEM),
            pl.BlockSpec(memory_space=pltpu.SMEM),
        ],
        out_specs=pl.BlockSpec(memory_space=pl.ANY),
        scratch_shapes=[
            pltpu.VMEM((2, m_per, N_SUB), COMM_DTYPE),
            pltpu.VMEM((2, m_per, N_SUB), COMM_DTYPE),
            pltpu.VMEM((2, m_per, N_SUB), COMM_DTYPE),
            pltpu.VMEM((2, m_per, N_SUB), COMM_DTYPE),
            pltpu.VMEM((m_per, N_SUB), jnp.float32),
            pltpu.SemaphoreType.DMA((4, 2)),
            pltpu.SemaphoreType.DMA((4, 2)),
            pltpu.SemaphoreType.DMA,
            pltpu.SemaphoreType.REGULAR,
            pltpu.SemaphoreType.REGULAR,
            pltpu.SemaphoreType.REGULAR,
            pltpu.SemaphoreType.REGULAR,
        ],
        compiler_params=pltpu.CompilerParams(
            collective_id=0, vmem_limit_bytes=50 * 1024 * 1024),
    )(x, w_mat, scale)


# device time: 332127 ns/iter; 1.0758x vs baseline; 1.0651x over previous
import jax
import jax.numpy as jnp
from jax import lax
from jax.experimental import pallas as pl
from jax.experimental.pallas import tpu as pltpu

N_DEV = 4
N_SUB = 1024
N_CHAIN = 2
COMM_DTYPE = jnp.bfloat16


def kernel(x, w_mat, scale_x, scale_w):
    m_total, _k = x.shape
    _k2, n = w_mat.shape
    m_per = m_total // N_DEV
    half = n // 2
    nb = half // N_SUB
    hops_per_ring = (nb // N_CHAIN) * (N_DEV - 1)

    x = x.astype(jnp.float8_e4m3fn)
    w_mat = w_mat.astype(jnp.float8_e5m2)
    scale = (scale_x * scale_w).reshape(1, 1)

    def body(x_ref, w_ref, scale_ref, out_ref,
             comm0, comm1, comm2, comm3, stage,
             send_sems, recv_sems, out_sem,
             ready0, ready1, ready2, ready3):
        d = lax.axis_index("i")
        right = lax.rem(d + 1, N_DEV)
        left = lax.rem(d + N_DEV - 1, N_DEV)

        barrier_sem = pltpu.get_barrier_semaphore()
        for nbr in (left, right):
            pl.semaphore_signal(barrier_sem, inc=1, device_id=(nbr,),
                                device_id_type=pl.DeviceIdType.MESH)
        pl.semaphore_wait(barrier_sem, 2)

        def partial(chunk_idx, col_off):
            xc = x_ref[pl.ds(chunk_idx * m_per, m_per), :]
            wc = w_ref[:, pl.ds(col_off, N_SUB)]
            return lax.dot_general(xc, wc, (((1,), (0,)), ((), ())),
                                   preferred_element_type=jnp.float32)

        dir_specs = [
            dict(tgt=right, upstream=left, base=0,
                 seed_c=lax.rem(d + N_DEV - 1, N_DEV),
                 in_c=lambda s: lax.rem(d + 2 * N_DEV - 2 - s, N_DEV)),
            dict(tgt=left, upstream=right, base=half,
                 seed_c=lax.rem(d + 1, N_DEV),
                 in_c=lambda s: lax.rem(d + 2 + s, N_DEV)),
        ]
        comms = [comm0, comm1, comm2, comm3]
        readys = [ready0, ready1, ready2, ready3]
        out_copies = []

        rings = []
        for di, ds_ in enumerate(dir_specs):
            for ch in range(N_CHAIN):
                r = di * N_CHAIN + ch
                blocks = list(range(ch, nb, N_CHAIN))
                rings.append(dict(
                    ds_, idx=r, dir=di, comm=comms[r], ready=readys[r],
                    tokens=[(b, s) for b in blocks for s in range(N_DEV - 1)],
                    H=0, rdma=None, p=None,
                ))
                pl.semaphore_signal(readys[r], inc=1,
                                    device_id=(ds_["upstream"],),
                                    device_id_type=pl.DeviceIdType.MESH)

        rings = [rings[0], rings[2], rings[1], rings[3]]

        def col(ring, blk):
            return ring["base"] + blk * N_SUB

        def seed(ring, blk):
            ring["comm"][ring["H"] % 2] = partial(
                ring["seed_c"], col(ring, blk)).astype(COMM_DTYPE)

        def start(ring):
            H = ring["H"]
            pl.semaphore_wait(ring["ready"], 1)
            rdma = pltpu.make_async_remote_copy(
                src_ref=ring["comm"].at[H % 2],
                dst_ref=ring["comm"].at[(H + 1) % 2],
                send_sem=send_sems.at[ring["idx"], H % 2],
                recv_sem=recv_sems.at[ring["idx"], (H + 1) % 2],
                device_id=(ring["tgt"],),
                device_id_type=pl.DeviceIdType.MESH,
            )
            rdma.start()
            ring["rdma"] = rdma

        def stash(ring, blk, s):
            ring["p"] = partial(ring["in_c"](s), col(ring, blk)).astype(
                COMM_DTYPE)

        def finish(ring, blk, s):
            H = ring["H"]
            rdma = ring["rdma"]
            rdma.wait_recv()
            if s < N_DEV - 2:
                ring["comm"][(H + 1) % 2] = (
                    ring["comm"][(H + 1) % 2] + ring["p"])
            else:
                acc = (ring["comm"][(H + 1) % 2].astype(jnp.float32)
                       + ring["p"].astype(jnp.float32))
                if out_copies:
                    out_copies[-1].wait()
                stage[...] = jnp.maximum(acc * scale_ref[0, 0], 0.0)
                cp = pltpu.make_async_copy(
                    stage,
                    out_ref.at[:, pl.ds(col(ring, blk), N_SUB)],
                    out_sem,
                )
                cp.start()
                out_copies.append(cp)
            rdma.wait_send()
            ring["H"] = H + 1
            if ring["H"] < hops_per_ring:
                pl.semaphore_signal(ring["ready"], inc=1,
                                    device_id=(ring["upstream"],),
                                    device_id_type=pl.DeviceIdType.MESH)

        for ring in rings:
            blk, s = ring["tokens"][0]
            seed(ring, blk)
            start(ring)
            stash(ring, blk, s)
        for t in range(hops_per_ring):
            for ring in rings:
                blk, s = ring["tokens"][t]
                finish(ring, blk, s)
                if t + 1 < hops_per_ring:
                    nblk, ns = ring["tokens"][t + 1]
                    if ns == 0:
                        seed(ring, nblk)
                    start(ring)
                    stash(ring, nblk, ns)
        out_copies[-1].wait()

    return pl.pallas_call(
        body,
        out_shape=jax.ShapeDtypeStruct((m_per, n), jnp.float32),
        in_specs=[
            pl.BlockSpec(memory_space=pltpu.VMEM),
            pl.BlockSpec(memory_space=pltpu.VMEM),
            pl.BlockSpec(memory_space=pltpu.SMEM),
        ],
        out_specs=pl.BlockSpec(memory_space=pl.ANY),
        scratch_shapes=[
            pltpu.VMEM((2, m_per, N_SUB), COMM_DTYPE),
            pltpu.VMEM((2, m_per, N_SUB), COMM_DTYPE),
            pltpu.VMEM((2, m_per, N_SUB), COMM_DTYPE),
            pltpu.VMEM((2, m_per, N_SUB), COMM_DTYPE),
            pltpu.VMEM((m_per, N_SUB), jnp.float32),
            pltpu.SemaphoreType.DMA((4, 2)),
            pltpu.SemaphoreType.DMA((4, 2)),
            pltpu.SemaphoreType.DMA,
            pltpu.SemaphoreType.REGULAR,
            pltpu.SemaphoreType.REGULAR,
            pltpu.SemaphoreType.REGULAR,
            pltpu.SemaphoreType.REGULAR,
        ],
        compiler_params=pltpu.CompilerParams(
            collective_id=0, vmem_limit_bytes=50 * 1024 * 1024),
    )(x, w_mat, scale)
